# baseline (speedup 1.0000x reference)
import functools

import jax
import jax.numpy as jnp
from jax import lax
from jax.experimental import pallas as pl
from jax.experimental.pallas import tpu as pltpu
from jax.experimental.pallas import tpu_sc as plsc

N = 10000
E = 320000
D = 128

NC = 2
NS = 16
NW = NC * NS
EDGES_PER_TILE = E // NW
CHUNK = 104
NCHUNK = 98
E_TILE = NCHUNK * CHUNK
N_PAD = 10112
ROWS_PER_TILE = N_PAD // NS

BM = 2000

NBUF = 2


def _segment_sum_sc(p, src2, dst3, zeros):
    mesh = plsc.VectorSubcoreMesh(core_axis_name="c", subcore_axis_name="s",
                                  num_cores=NC, num_subcores=NS)

    @functools.partial(
        pl.kernel,
        out_type=jax.ShapeDtypeStruct((NC * N_PAD, D), jnp.float32),
        mesh=mesh,
        scratch_types=[
            pltpu.VMEM_SHARED((N_PAD, D), jnp.float32),
            pltpu.VMEM((E_TILE,), jnp.int32),
            pltpu.VMEM((NCHUNK, CHUNK), jnp.int32),
            pltpu.VMEM((NBUF, CHUNK, D), jnp.float32),
            pltpu.SemaphoreType.DMA,
        ],
    )
    def k(p_hbm, src_hbm, dst_hbm, zeros_hbm, out_hbm,
          acc, sidx, didx, rows, gsem):
        cid = lax.axis_index("c")
        sid = lax.axis_index("s")
        wid = cid * NS + sid
        row0 = sid * ROWS_PER_TILE
        pltpu.sync_copy(src_hbm.at[wid], sidx)
        pltpu.sync_copy(dst_hbm.at[wid], didx)
        pltpu.sync_copy(zeros_hbm, acc.at[pl.ds(row0, ROWS_PER_TILE)])
        plsc.subcore_barrier()

        for b in range(NBUF):
            pltpu.async_copy(p_hbm.at[sidx.at[pl.ds(b * CHUNK, CHUNK)]],
                             rows.at[b], gsem)

        def step(j, b, prefetch):
            pltpu.make_async_copy(p_hbm.at[sidx.at[pl.ds(0, CHUNK)]],
                                  rows.at[b], gsem).wait()
            pltpu.sync_copy(rows.at[b], acc.at[didx.at[j]], add=True)
            if prefetch:
                jn = j + NBUF
                pltpu.async_copy(p_hbm.at[sidx.at[pl.ds(jn * CHUNK, CHUNK)]],
                                 rows.at[b], gsem)

        def body(j, carry):
            step(j, lax.rem(j, NBUF), True)
            return carry

        lax.fori_loop(0, NCHUNK - NBUF, body, 0)
        for jj in range(NCHUNK - NBUF, NCHUNK):
            step(jj, jj % NBUF, False)

        plsc.subcore_barrier()
        pltpu.sync_copy(acc.at[pl.ds(row0, ROWS_PER_TILE)],
                        out_hbm.at[pl.ds(cid * N_PAD + row0, ROWS_PER_TILE)])

    return k(p, src2, dst3, zeros)


def _mm(h, W):
    def body(h_ref, w_ref, o_ref):
        o_ref[...] = jnp.dot(h_ref[...], w_ref[...],
                             preferred_element_type=jnp.float32)

    return pl.pallas_call(
        body,
        grid=(N // BM,),
        in_specs=[pl.BlockSpec((BM, D), lambda i: (i, 0)),
                  pl.BlockSpec((D, D), lambda i: (0, 0))],
        out_specs=pl.BlockSpec((BM, D), lambda i: (i, 0)),
        out_shape=jax.ShapeDtypeStruct((N, D), jnp.float32),
    )(h, W)


def _elu(t):
    return jnp.where(t > 0, t, jnp.exp(jnp.minimum(t, 0.0)) - 1.0)


def _fused_elu_mm(p, s0, s1, b, W):
    def body(p_ref, s0_ref, s1_ref, b_ref, w_ref, o_ref):
        t = p_ref[...] + s0_ref[...] + s1_ref[...] + b_ref[...]
        o_ref[...] = jnp.dot(_elu(t), w_ref[...],
                             preferred_element_type=jnp.float32)

    return pl.pallas_call(
        body,
        grid=(N // BM,),
        in_specs=[pl.BlockSpec((BM, D), lambda i: (i, 0)),
                  pl.BlockSpec((BM, D), lambda i: (i, 0)),
                  pl.BlockSpec((BM, D), lambda i: (i, 0)),
                  pl.BlockSpec((1, D), lambda i: (0, 0)),
                  pl.BlockSpec((D, D), lambda i: (0, 0))],
        out_specs=pl.BlockSpec((BM, D), lambda i: (i, 0)),
        out_shape=jax.ShapeDtypeStruct((N, D), jnp.float32),
    )(p, s0, s1, b, W)


def _fused_elu(p, s0, s1, b):
    def body(p_ref, s0_ref, s1_ref, b_ref, o_ref):
        t = p_ref[...] + s0_ref[...] + s1_ref[...] + b_ref[...]
        o_ref[...] = _elu(t)

    return pl.pallas_call(
        body,
        grid=(N // BM,),
        in_specs=[pl.BlockSpec((BM, D), lambda i: (i, 0)),
                  pl.BlockSpec((BM, D), lambda i: (i, 0)),
                  pl.BlockSpec((BM, D), lambda i: (i, 0)),
                  pl.BlockSpec((1, D), lambda i: (0, 0))],
        out_specs=pl.BlockSpec((BM, D), lambda i: (i, 0)),
        out_shape=jax.ShapeDtypeStruct((N, D), jnp.float32),
    )(p, s0, s1, b)


def kernel(x, edge_index, x_param, W0, b0, W1, b1, W2, b2):
    pad = E_TILE - EDGES_PER_TILE
    src = jnp.pad(edge_index[0].reshape(NW, EDGES_PER_TILE), ((0, 0), (0, pad)))
    dst = jnp.pad(edge_index[1].reshape(NW, EDGES_PER_TILE), ((0, 0), (0, pad)),
                  constant_values=N).reshape(NW, NCHUNK, CHUNK)

    zeros = jnp.zeros((ROWS_PER_TILE, D), jnp.float32)
    b0r, b1r, b2r = (b.reshape(1, D) for b in (b0, b1, b2))

    p = _mm(x_param, W0)
    s = _segment_sum_sc(p, src, dst, zeros)
    p = _fused_elu_mm(p, s[:N], s[N_PAD:N_PAD + N], b0r, W1)
    s = _segment_sum_sc(p, src, dst, zeros)
    p = _fused_elu_mm(p, s[:N], s[N_PAD:N_PAD + N], b1r, W2)
    s = _segment_sum_sc(p, src, dst, zeros)
    return _fused_elu(p, s[:N], s[N_PAD:N_PAD + N], b2r)


# speedup vs baseline: 1.5685x; 1.5685x over previous
import functools

import jax
import jax.numpy as jnp
from jax import lax
from jax.experimental import pallas as pl
from jax.experimental.pallas import tpu as pltpu
from jax.experimental.pallas import tpu_sc as plsc

N = 10000
E = 320000
D = 128

NC = 2
NS = 16
NW = NC * NS
EDGES_PER_TILE = E // NW
CHUNK = 96
NCHUNK = 105
E_TILE = NCHUNK * CHUNK
N_PAD = 10112
ROWS_PER_TILE = N_PAD // NS

BM = 2000

NBUF = 2


def _segment_sum_sc(p, src2, dst3, zeros):
    mesh = plsc.VectorSubcoreMesh(core_axis_name="c", subcore_axis_name="s",
                                  num_cores=NC, num_subcores=NS)

    @functools.partial(
        pl.kernel,
        out_type=jax.ShapeDtypeStruct((NC * N_PAD, D), jnp.float32),
        mesh=mesh,
        scratch_types=[
            pltpu.VMEM_SHARED((N_PAD, D), jnp.float32),
            pltpu.VMEM((E_TILE,), jnp.int32),
            pltpu.VMEM((NCHUNK, CHUNK), jnp.int32),
            pltpu.VMEM((NBUF, CHUNK, D), jnp.float32),
            pltpu.SemaphoreType.DMA,
        ],
    )
    def k(p_hbm, src_hbm, dst_hbm, zeros_hbm, out_hbm,
          acc, sidx, didx, rows, gsem):
        cid = lax.axis_index("c")
        sid = lax.axis_index("s")
        wid = cid * NS + sid
        row0 = sid * ROWS_PER_TILE
        pltpu.sync_copy(src_hbm.at[wid], sidx)
        pltpu.sync_copy(dst_hbm.at[wid], didx)
        pltpu.sync_copy(zeros_hbm, acc.at[pl.ds(row0, ROWS_PER_TILE)])
        plsc.subcore_barrier()

        for b in range(NBUF):
            pltpu.async_copy(p_hbm.at[sidx.at[pl.ds(b * CHUNK, CHUNK)]],
                             rows.at[b], gsem)

        def step(j, b, prefetch):
            pltpu.make_async_copy(p_hbm.at[sidx.at[pl.ds(0, CHUNK)]],
                                  rows.at[b], gsem).wait()
            pltpu.sync_copy(rows.at[b], acc.at[didx.at[j]], add=True)
            if prefetch:
                jn = j + NBUF
                pltpu.async_copy(p_hbm.at[sidx.at[pl.ds(jn * CHUNK, CHUNK)]],
                                 rows.at[b], gsem)

        def body(j, carry):
            step(j, lax.rem(j, NBUF), True)
            return carry

        lax.fori_loop(0, NCHUNK - NBUF, body, 0)
        for jj in range(NCHUNK - NBUF, NCHUNK):
            step(jj, jj % NBUF, False)

        plsc.subcore_barrier()
        pltpu.sync_copy(acc.at[pl.ds(row0, ROWS_PER_TILE)],
                        out_hbm.at[pl.ds(cid * N_PAD + row0, ROWS_PER_TILE)])

    return k(p, src2, dst3, zeros)


def _mm(h, W):
    def body(h_ref, w_ref, o_ref):
        o_ref[...] = jnp.dot(h_ref[...], w_ref[...],
                             preferred_element_type=jnp.float32)

    return pl.pallas_call(
        body,
        grid=(N // BM,),
        in_specs=[pl.BlockSpec((BM, D), lambda i: (i, 0)),
                  pl.BlockSpec((D, D), lambda i: (0, 0))],
        out_specs=pl.BlockSpec((BM, D), lambda i: (i, 0)),
        out_shape=jax.ShapeDtypeStruct((N, D), jnp.float32),
    )(h, W)


def _elu(t):
    return jnp.where(t > 0, t, jnp.exp(jnp.minimum(t, 0.0)) - 1.0)


def _fused_elu_mm(p, s0, s1, b, W):
    def body(p_ref, s0_ref, s1_ref, b_ref, w_ref, o_ref):
        t = p_ref[...] + s0_ref[...] + s1_ref[...] + b_ref[...]
        o_ref[...] = jnp.dot(_elu(t), w_ref[...],
                             preferred_element_type=jnp.float32)

    return pl.pallas_call(
        body,
        grid=(N // BM,),
        in_specs=[pl.BlockSpec((BM, D), lambda i: (i, 0)),
                  pl.BlockSpec((BM, D), lambda i: (i, 0)),
                  pl.BlockSpec((BM, D), lambda i: (i, 0)),
                  pl.BlockSpec((1, D), lambda i: (0, 0)),
                  pl.BlockSpec((D, D), lambda i: (0, 0))],
        out_specs=pl.BlockSpec((BM, D), lambda i: (i, 0)),
        out_shape=jax.ShapeDtypeStruct((N, D), jnp.float32),
    )(p, s0, s1, b, W)


def _fused_elu(p, s0, s1, b):
    def body(p_ref, s0_ref, s1_ref, b_ref, o_ref):
        t = p_ref[...] + s0_ref[...] + s1_ref[...] + b_ref[...]
        o_ref[...] = _elu(t)

    return pl.pallas_call(
        body,
        grid=(N // BM,),
        in_specs=[pl.BlockSpec((BM, D), lambda i: (i, 0)),
                  pl.BlockSpec((BM, D), lambda i: (i, 0)),
                  pl.BlockSpec((BM, D), lambda i: (i, 0)),
                  pl.BlockSpec((1, D), lambda i: (0, 0))],
        out_specs=pl.BlockSpec((BM, D), lambda i: (i, 0)),
        out_shape=jax.ShapeDtypeStruct((N, D), jnp.float32),
    )(p, s0, s1, b)


def kernel(x, edge_index, x_param, W0, b0, W1, b1, W2, b2):
    pad = E_TILE - EDGES_PER_TILE
    src = jnp.pad(edge_index[0].reshape(NW, EDGES_PER_TILE), ((0, 0), (0, pad)))
    dst = jnp.pad(edge_index[1].reshape(NW, EDGES_PER_TILE), ((0, 0), (0, pad)),
                  constant_values=N).reshape(NW, NCHUNK, CHUNK)

    zeros = jnp.zeros((ROWS_PER_TILE, D), jnp.float32)
    b0r, b1r, b2r = (b.reshape(1, D) for b in (b0, b1, b2))

    p = _mm(x_param, W0)
    s = _segment_sum_sc(p, src, dst, zeros)
    p = _fused_elu_mm(p, s[:N], s[N_PAD:N_PAD + N], b0r, W1)
    s = _segment_sum_sc(p, src, dst, zeros)
    p = _fused_elu_mm(p, s[:N], s[N_PAD:N_PAD + N], b1r, W2)
    s = _segment_sum_sc(p, src, dst, zeros)
    return _fused_elu(p, s[:N], s[N_PAD:N_PAD + N], b2r)


# speedup vs baseline: 1.6229x; 1.0347x over previous
import functools

import jax
import jax.numpy as jnp
from jax import lax
from jax.experimental import pallas as pl
from jax.experimental.pallas import tpu as pltpu
from jax.experimental.pallas import tpu_sc as plsc

N = 10000
E = 320000
D = 128

NC = 2
NS = 16
NW = NC * NS
EDGES_PER_TILE = E // NW
CHUNK = 96
NCHUNK = 105
E_TILE = NCHUNK * CHUNK
N_PAD = 10112
ROWS_PER_TILE = N_PAD // NS

BM = 2000

NBUF = 2


def _segment_sum_sc(p, src2, dst3, zeros):
    mesh = plsc.VectorSubcoreMesh(core_axis_name="c", subcore_axis_name="s",
                                  num_cores=NC, num_subcores=NS)

    @functools.partial(
        pl.kernel,
        out_type=jax.ShapeDtypeStruct((NC, N_PAD, D), jnp.float32),
        mesh=mesh,
        scratch_types=[
            pltpu.VMEM_SHARED((N_PAD, D), jnp.float32),
            pltpu.VMEM((E_TILE,), jnp.int32),
            pltpu.VMEM((NCHUNK, CHUNK), jnp.int32),
            pltpu.VMEM((NBUF, CHUNK, D), jnp.float32),
            pltpu.SemaphoreType.DMA,
        ],
    )
    def k(p_hbm, src_hbm, dst_hbm, zeros_hbm, out_hbm,
          acc, sidx, didx, rows, gsem):
        cid = lax.axis_index("c")
        sid = lax.axis_index("s")
        wid = cid * NS + sid
        row0 = sid * ROWS_PER_TILE
        pltpu.sync_copy(src_hbm.at[wid], sidx)
        pltpu.sync_copy(dst_hbm.at[wid], didx)
        for b in range(NBUF):
            pltpu.async_copy(p_hbm.at[sidx.at[pl.ds(b * CHUNK, CHUNK)]],
                             rows.at[b], gsem)
        pltpu.sync_copy(zeros_hbm, acc.at[pl.ds(row0, ROWS_PER_TILE)])
        plsc.subcore_barrier()

        def step(j, b, prefetch):
            pltpu.make_async_copy(p_hbm.at[sidx.at[pl.ds(0, CHUNK)]],
                                  rows.at[b], gsem).wait()
            pltpu.sync_copy(rows.at[b], acc.at[didx.at[j]], add=True)
            if prefetch:
                jn = j + NBUF
                pltpu.async_copy(p_hbm.at[sidx.at[pl.ds(jn * CHUNK, CHUNK)]],
                                 rows.at[b], gsem)

        def body(j, carry):
            step(j, lax.rem(j, NBUF), True)
            return carry

        lax.fori_loop(0, NCHUNK - NBUF, body, 0)
        for jj in range(NCHUNK - NBUF, NCHUNK):
            step(jj, jj % NBUF, False)

        plsc.subcore_barrier()
        pltpu.sync_copy(acc.at[pl.ds(row0, ROWS_PER_TILE)],
                        out_hbm.at[cid, pl.ds(row0, ROWS_PER_TILE)])

    return k(p, src2, dst3, zeros)


def _mm(h, W):
    def body(h_ref, w_ref, o_ref):
        o_ref[...] = jnp.dot(h_ref[...], w_ref[...],
                             preferred_element_type=jnp.float32)

    return pl.pallas_call(
        body,
        grid=(N // BM,),
        in_specs=[pl.BlockSpec((BM, D), lambda i: (i, 0)),
                  pl.BlockSpec((D, D), lambda i: (0, 0))],
        out_specs=pl.BlockSpec((BM, D), lambda i: (i, 0)),
        out_shape=jax.ShapeDtypeStruct((N, D), jnp.float32),
    )(h, W)


def _elu(t):
    return jnp.where(t > 0, t, jnp.exp(jnp.minimum(t, 0.0)) - 1.0)


def _fused_elu_mm(p, s, b, W):
    def body(p_ref, s0_ref, s1_ref, b_ref, w_ref, o_ref):
        t = p_ref[...] + s0_ref[0] + s1_ref[0] + b_ref[...]
        o_ref[...] = jnp.dot(_elu(t), w_ref[...],
                             preferred_element_type=jnp.float32)

    return pl.pallas_call(
        body,
        grid=(N // BM,),
        in_specs=[pl.BlockSpec((BM, D), lambda i: (i, 0)),
                  pl.BlockSpec((1, BM, D), lambda i: (0, i, 0)),
                  pl.BlockSpec((1, BM, D), lambda i: (1, i, 0)),
                  pl.BlockSpec((1, D), lambda i: (0, 0)),
                  pl.BlockSpec((D, D), lambda i: (0, 0))],
        out_specs=pl.BlockSpec((BM, D), lambda i: (i, 0)),
        out_shape=jax.ShapeDtypeStruct((N, D), jnp.float32),
    )(p, s, s, b, W)


def _fused_elu(p, s, b):
    def body(p_ref, s0_ref, s1_ref, b_ref, o_ref):
        t = p_ref[...] + s0_ref[0] + s1_ref[0] + b_ref[...]
        o_ref[...] = _elu(t)

    return pl.pallas_call(
        body,
        grid=(N // BM,),
        in_specs=[pl.BlockSpec((BM, D), lambda i: (i, 0)),
                  pl.BlockSpec((1, BM, D), lambda i: (0, i, 0)),
                  pl.BlockSpec((1, BM, D), lambda i: (1, i, 0)),
                  pl.BlockSpec((1, D), lambda i: (0, 0))],
        out_specs=pl.BlockSpec((BM, D), lambda i: (i, 0)),
        out_shape=jax.ShapeDtypeStruct((N, D), jnp.float32),
    )(p, s, s, b)


def kernel(x, edge_index, x_param, W0, b0, W1, b1, W2, b2):
    pad = E_TILE - EDGES_PER_TILE
    src = jnp.pad(edge_index[0].reshape(NW, EDGES_PER_TILE), ((0, 0), (0, pad)))
    dst = jnp.pad(edge_index[1].reshape(NW, EDGES_PER_TILE), ((0, 0), (0, pad)),
                  constant_values=N).reshape(NW, NCHUNK, CHUNK)

    zeros = jnp.zeros((ROWS_PER_TILE, D), jnp.float32)
    b0r, b1r, b2r = (b.reshape(1, D) for b in (b0, b1, b2))

    p = _mm(x_param, W0)
    s = _segment_sum_sc(p, src, dst, zeros)
    p = _fused_elu_mm(p, s, b0r, W1)
    s = _segment_sum_sc(p, src, dst, zeros)
    p = _fused_elu_mm(p, s, b1r, W2)
    s = _segment_sum_sc(p, src, dst, zeros)
    return _fused_elu(p, s, b2r)
